# baseline (speedup 1.0000x reference)
import jax
import jax.numpy as jnp
import numpy as np
from jax.experimental import pallas as pl

_FEAT_STRIDE = 16
_SCALES = np.array([8.0, 16.0, 32.0])
_RATIOS = np.array([0.5, 1.0, 2.0])


def _whctrs(anchor):
    w = anchor[2] - anchor[0] + 1
    h = anchor[3] - anchor[1] + 1
    return w, h, anchor[0] + 0.5 * (w - 1), anchor[1] + 0.5 * (h - 1)


def _mkanchors(ws, hs, x_ctr, y_ctr):
    ws = ws[:, None]
    hs = hs[:, None]
    return np.hstack((x_ctr - 0.5 * (ws - 1), y_ctr - 0.5 * (hs - 1),
                      x_ctr + 0.5 * (ws - 1), y_ctr + 0.5 * (hs - 1)))


def _base_anchors():
    base = np.array([1.0, 1.0, float(_FEAT_STRIDE), float(_FEAT_STRIDE)]) - 1
    w, h, xc, yc = _whctrs(base)
    size_ratios = (w * h) / _RATIOS
    ws = np.round(np.sqrt(size_ratios))
    hs = np.round(ws * _RATIOS)
    ratio_anchors = _mkanchors(ws, hs, xc, yc)
    rows = []
    for i in range(ratio_anchors.shape[0]):
        w, h, xc, yc = _whctrs(ratio_anchors[i, :])
        rows.append(_mkanchors(w * _SCALES, h * _SCALES, xc, yc))
    return np.vstack(rows)


def _body(scores_ref, base_ref, fg_ref, anc_ref):
    fg_ref[...] = scores_ref[...]
    k = jax.lax.broadcasted_iota(jnp.int32, (4096, 36), 0)
    j = jax.lax.broadcasted_iota(jnp.int32, (4096, 36), 1)
    x = jnp.bitwise_and(k, 63)
    y = jax.lax.shift_right_logical(k, 6)
    shift = jnp.where(jnp.bitwise_and(j, 1) == 0, x, y).astype(jnp.float32)
    anc_ref[0] = base_ref[0] + shift * float(_FEAT_STRIDE)


def kernel(scores, bbox_deltas, im_info, cfg_key):
    B = scores.shape[0]
    A = 9
    H, W = scores.shape[2], scores.shape[3]
    K = H * W
    base36 = jnp.asarray(_base_anchors().reshape(1, 4 * A), dtype=jnp.float32)

    scores_fg, anc3 = pl.pallas_call(
        _body,
        grid=(B,),
        in_specs=[
            pl.BlockSpec((1, A, H, W), lambda b: (b, 1, 0, 0)),
            pl.BlockSpec((1, 4 * A), lambda b: (0, 0)),
        ],
        out_specs=[
            pl.BlockSpec((1, A, H, W), lambda b: (b, 0, 0, 0)),
            pl.BlockSpec((1, K, 4 * A), lambda b: (b, 0, 0)),
        ],
        out_shape=[
            jax.ShapeDtypeStruct((B, A, H, W), jnp.float32),
            jax.ShapeDtypeStruct((B, K, 4 * A), jnp.float32),
        ],
    )(scores, base36)

    anchors = anc3.reshape(B, K * A, 4)
    return (scores_fg, bbox_deltas, im_info, anchors)


# speedup vs baseline: 1.6593x; 1.6593x over previous
import jax
import jax.numpy as jnp
import numpy as np
from jax.experimental import pallas as pl

_FEAT_STRIDE = 16
_SCALES = np.array([8.0, 16.0, 32.0])
_RATIOS = np.array([0.5, 1.0, 2.0])


def _whctrs(anchor):
    w = anchor[2] - anchor[0] + 1
    h = anchor[3] - anchor[1] + 1
    return w, h, anchor[0] + 0.5 * (w - 1), anchor[1] + 0.5 * (h - 1)


def _mkanchors(ws, hs, x_ctr, y_ctr):
    ws = ws[:, None]
    hs = hs[:, None]
    return np.hstack((x_ctr - 0.5 * (ws - 1), y_ctr - 0.5 * (hs - 1),
                      x_ctr + 0.5 * (ws - 1), y_ctr + 0.5 * (hs - 1)))


def _base_anchors():
    base = np.array([1.0, 1.0, float(_FEAT_STRIDE), float(_FEAT_STRIDE)]) - 1
    w, h, xc, yc = _whctrs(base)
    size_ratios = (w * h) / _RATIOS
    ws = np.round(np.sqrt(size_ratios))
    hs = np.round(ws * _RATIOS)
    ratio_anchors = _mkanchors(ws, hs, xc, yc)
    rows = []
    for i in range(ratio_anchors.shape[0]):
        w, h, xc, yc = _whctrs(ratio_anchors[i, :])
        rows.append(_mkanchors(w * _SCALES, h * _SCALES, xc, yc))
    return np.vstack(rows)


def _pattern_table():
    base36 = _base_anchors().reshape(36).astype(np.float64)
    pl_idx = (np.arange(9)[:, None] * 128 + np.arange(128)[None, :])
    j = pl_idx % 36
    k_off = pl_idx // 36
    tab = np.stack([
        k_off.astype(np.float64),
        base36[j],
        (j % 2 == 0).astype(np.float64),
    ])
    return tab.reshape(27, 128).astype(np.float32)


def _body(scores_ref, tab_ref, fg_ref, anc_ref):
    fg_ref[...] = scores_ref[0]

    tab = tab_ref[...].reshape(3, 9, 128)
    k_off = tab[0][None]
    base = tab[1][None]
    is_x = tab[2][None]
    q = jax.lax.broadcasted_iota(jnp.int32, (128, 9, 128), 0).astype(jnp.float32)
    k = q * 32.0 + k_off
    y = jnp.floor(k * (1.0 / 64.0))
    x = k - y * 64.0
    shift = jnp.where(is_x > 0.5, x, y)
    anc_ref[0] = base + shift * float(_FEAT_STRIDE)


def kernel(scores, bbox_deltas, im_info, cfg_key):
    B = scores.shape[0]
    A = 9
    H, W = scores.shape[2], scores.shape[3]
    K = H * W

    scores4 = scores.reshape(B, 2, (A * K) // 128, 128)
    tab = jnp.asarray(_pattern_table())

    fg, anc = pl.pallas_call(
        _body,
        grid=(B,),
        in_specs=[
            pl.BlockSpec((1, 1, (A * K) // 128, 128), lambda b: (b, 1, 0, 0)),
            pl.BlockSpec((27, 128), lambda b: (0, 0)),
        ],
        out_specs=[
            pl.BlockSpec((1, (A * K) // 128, 128), lambda b: (b, 0, 0)),
            pl.BlockSpec((1, 128, 9, 128), lambda b: (b, 0, 0, 0)),
        ],
        out_shape=[
            jax.ShapeDtypeStruct((B, (A * K) // 128, 128), jnp.float32),
            jax.ShapeDtypeStruct((B, 128, 9, 128), jnp.float32),
        ],
    )(scores4, tab)

    scores_fg = fg.reshape(B, A, H, W)
    anchors = anc.reshape(B, K * A, 4)
    return (scores_fg, bbox_deltas, im_info, anchors)


# speedup vs baseline: 2.1036x; 1.2677x over previous
import jax
import jax.numpy as jnp
from jax.experimental import pallas as pl
from jax.experimental.pallas import tpu as pltpu

_FEAT_STRIDE = 16.0


def _anchor_pattern():
    r = jax.lax.broadcasted_iota(jnp.int32, (1152, 128), 0)
    l = jax.lax.broadcasted_iota(jnp.int32, (1152, 128), 1)
    i = (r * 128 + l).astype(jnp.float32)
    k = jnp.floor((i + 0.5) * (1.0 / 36.0))
    j = i - 36.0 * k
    a = jnp.floor((j + 0.5) * 0.25)
    c = j - 4.0 * a
    ri = jnp.floor((a + 0.5) * (1.0 / 3.0))
    si = a - 3.0 * ri
    ws = jnp.where(ri < 0.5, 23.0, jnp.where(ri < 1.5, 16.0, 11.0))
    hs = jnp.where(ri < 0.5, 12.0, jnp.where(ri < 1.5, 16.0, 22.0))
    sc = jnp.where(si < 0.5, 8.0, jnp.where(si < 1.5, 16.0, 32.0))
    hw = 0.5 * (ws * sc - 1.0)
    hh = 0.5 * (hs * sc - 1.0)
    base = jnp.where(c < 0.5, 7.5 - hw,
                     jnp.where(c < 1.5, 7.5 - hh,
                               jnp.where(c < 2.5, 7.5 + hw, 7.5 + hh)))
    y = jnp.floor(k * (1.0 / 64.0))
    x = k - 64.0 * y
    c_even = jnp.logical_or(c < 0.5, jnp.abs(c - 2.0) < 0.5)
    return base + _FEAT_STRIDE * jnp.where(c_even, x, y)


def _body(scores_ref, fg_ref, anc_ref, pat_ref):
    @pl.when(pl.program_id(0) == 0)
    def _():
        pat_ref[...] = _anchor_pattern()

    fg_ref[...] = scores_ref[...]
    anc_ref[0] = pat_ref[...]


def kernel(scores, bbox_deltas, im_info, cfg_key):
    B = scores.shape[0]
    A = 9
    H, W = scores.shape[2], scores.shape[3]
    K = H * W

    fg, anc = pl.pallas_call(
        _body,
        grid=(B,),
        in_specs=[
            pl.BlockSpec((1, A, H, W), lambda b: (b, 1, 0, 0)),
        ],
        out_specs=[
            pl.BlockSpec((1, A, H, W), lambda b: (b, 0, 0, 0)),
            pl.BlockSpec((1, (K * A * 4) // 128, 128), lambda b: (b, 0, 0)),
        ],
        out_shape=[
            jax.ShapeDtypeStruct((B, A, H, W), jnp.float32),
            jax.ShapeDtypeStruct((B, (K * A * 4) // 128, 128), jnp.float32),
        ],
        scratch_shapes=[pltpu.VMEM(((K * A * 4) // 128, 128), jnp.float32)],
        compiler_params=pltpu.CompilerParams(
            dimension_semantics=("arbitrary",),
        ),
    )(scores)

    anchors = anc.reshape(B, K * A, 4)
    return (fg, bbox_deltas, im_info, anchors)


# speedup vs baseline: 7.0334x; 3.3436x over previous
import jax
import jax.numpy as jnp
from jax.experimental import pallas as pl
from jax.experimental.pallas import tpu as pltpu

_FEAT_STRIDE = 16.0


def _anchor_pattern():
    r = jax.lax.broadcasted_iota(jnp.int32, (1152, 128), 0)
    l = jax.lax.broadcasted_iota(jnp.int32, (1152, 128), 1)
    rf = r.astype(jnp.float32)
    g = jnp.floor(rf * 0.25)
    c = rf - 4.0 * g
    n = g * 128.0 + l.astype(jnp.float32)
    k = jnp.floor((n + 0.5) * (1.0 / 9.0))
    a = n - 9.0 * k
    ri = jnp.floor((a + 0.5) * (1.0 / 3.0))
    si = a - 3.0 * ri
    ws = jnp.where(ri < 0.5, 23.0, jnp.where(ri < 1.5, 16.0, 11.0))
    hs = jnp.where(ri < 0.5, 12.0, jnp.where(ri < 1.5, 16.0, 22.0))
    sc = jnp.where(si < 0.5, 8.0, jnp.where(si < 1.5, 16.0, 32.0))
    hw = 0.5 * (ws * sc - 1.0)
    hh = 0.5 * (hs * sc - 1.0)
    base = jnp.where(c < 0.5, 7.5 - hw,
                     jnp.where(c < 1.5, 7.5 - hh,
                               jnp.where(c < 2.5, 7.5 + hw, 7.5 + hh)))
    y = jnp.floor(k * (1.0 / 64.0))
    x = k - 64.0 * y
    c_even = jnp.logical_or(c < 0.5, jnp.abs(c - 2.0) < 0.5)
    return base + _FEAT_STRIDE * jnp.where(c_even, x, y)


def _body(scores_ref, fg_ref, anc_ref, pat_ref):
    @pl.when(pl.program_id(0) == 0)
    def _():
        pat_ref[...] = _anchor_pattern()

    fg_ref[...] = scores_ref[...]
    anc_ref[0] = pat_ref[...]


def kernel(scores, bbox_deltas, im_info, cfg_key):
    B = scores.shape[0]
    A = 9
    H, W = scores.shape[2], scores.shape[3]
    K = H * W

    fg, anc = pl.pallas_call(
        _body,
        grid=(B,),
        in_specs=[
            pl.BlockSpec((1, A, H, W), lambda b: (b, 1, 0, 0)),
        ],
        out_specs=[
            pl.BlockSpec((1, A, H, W), lambda b: (b, 0, 0, 0)),
            pl.BlockSpec((1, (K * A * 4) // 128, 128), lambda b: (b, 0, 0)),
        ],
        out_shape=[
            jax.ShapeDtypeStruct((B, A, H, W), jnp.float32),
            jax.ShapeDtypeStruct((B, (K * A * 4) // 128, 128), jnp.float32),
        ],
        scratch_shapes=[pltpu.VMEM(((K * A * 4) // 128, 128), jnp.float32)],
        compiler_params=pltpu.CompilerParams(
            dimension_semantics=("arbitrary",),
        ),
    )(scores)

    anchors = (anc.reshape(B, (K * A) // 128, 4, 128)
               .transpose(0, 1, 3, 2)
               .reshape(B, K * A, 4))
    return (fg, bbox_deltas, im_info, anchors)
